# Initial kernel scaffold; baseline (speedup 1.0000x reference)
import jax
import jax.numpy as jnp
from jax.experimental import pallas as pl


def kernel(stones, player, winners, actions, edge_index, params):
    raise NotImplementedError("write your pallas kernel here")


# speedup vs baseline: 13.8258x; 13.8258x over previous
import jax
import jax.numpy as jnp
from jax import lax
from jax.experimental import pallas as pl
from jax.experimental.pallas import tpu as pltpu

_B = 64
_S = 19
_N2 = _S * _S
_N = _B * _N2
_H = 64
_STEPS = 8
_BC = 8
_R = _BC * _N2
_NEG = -1e30


def _main_kernel(stones_ref, player_ref, rc_ref, ms_ref, mp_ref,
                 pw2_ref, pw3_ref, pw4_ref, pb2_ref, pb3_ref, pb4_ref,
                 w1ab_ref, mb1_ref, w2p_ref, w3p_ref, w4p_ref,
                 mb2p_ref, mb3p_ref, mb4p_ref,
                 qw1_ref, qw2_ref, qw3_ref, qw4_ref,
                 qb1_ref, qb2_ref, qb3_ref, qb4_ref,
                 lw_ref, lb_ref, wh_ref, bh_ref,
                 pol_ref, vs_ref):
    f32 = jnp.float32

    def mm(a, b):
        return jnp.dot(a, b, preferred_element_type=f32)

    s_col = stones_ref[...]
    p_col = player_ref[...]
    base = rc_ref[...]
    for k in range(3):
        base = base + (s_col == k).astype(f32) * ms_ref[k:k + 1, :]
    for k in range(2):
        base = base + (p_col == k).astype(f32) * mp_ref[k:k + 1, :]
    y = jax.nn.relu(base)
    y = jax.nn.relu(mm(y, pw2_ref[...]) + pb2_ref[...])
    y = jax.nn.relu(mm(y, pw3_ref[...]) + pb3_ref[...])
    x0 = mm(y, pw4_ref[...]) + pb4_ref[...]

    pos = lax.broadcasted_iota(jnp.int32, (_R, 1), 0) % _N2
    cp = pos % _S
    m_n = (pos >= _S).astype(f32)
    m_s = (pos < _N2 - _S).astype(f32)
    m_w = (cp >= 1).astype(f32)
    m_e = (cp <= _S - 2).astype(f32)

    z19 = jnp.zeros((_S, _H), f32)
    z1 = jnp.zeros((1, _H), f32)

    w1ab = w1ab_ref[...]
    mb1 = mb1_ref[...]
    w2p, w3p, w4p = w2p_ref[...], w3p_ref[...], w4p_ref[...]
    mb2p, mb3p, mb4p = mb2p_ref[...], mb3p_ref[...], mb4p_ref[...]
    qw1, qw2, qw3, qw4 = qw1_ref[...], qw2_ref[...], qw3_ref[...], qw4_ref[...]
    qb1, qb2, qb3, qb4 = qb1_ref[...], qb2_ref[...], qb3_ref[...], qb4_ref[...]
    lw, lb = lw_ref[...], lb_ref[...]
    wh, bh = wh_ref[...], bh_ref[...]

    x = x0
    h = jnp.zeros((_R, _H), f32)
    c = jnp.zeros((_R, _H), f32)

    for s in range(_STEPS):
        ab = mm(x, w1ab)
        a_part = ab[:, :_H]
        b_part = ab[:, _H:] + mb1
        a_n = jnp.concatenate([z19, a_part[:-_S]], axis=0)
        a_s = jnp.concatenate([a_part[_S:], z19], axis=0)
        a_w = jnp.concatenate([z1, a_part[:-1]], axis=0)
        a_e = jnp.concatenate([a_part[1:], z1], axis=0)
        h1n = jax.nn.relu(a_n + b_part)
        h1s = jax.nn.relu(a_s + b_part)
        h1w = jax.nn.relu(a_w + b_part)
        h1e = jax.nn.relu(a_e + b_part)
        ns = jnp.concatenate([h1n, h1s], axis=1)
        ns = jax.nn.relu(mm(ns, w2p) + mb2p)
        ns = jax.nn.relu(mm(ns, w3p) + mb3p)
        ns = mm(ns, w4p) + mb4p
        we = jnp.concatenate([h1w, h1e], axis=1)
        we = jax.nn.relu(mm(we, w2p) + mb2p)
        we = jax.nn.relu(mm(we, w3p) + mb3p)
        we = mm(we, w4p) + mb4p
        agg = (m_n * ns[:, :_H] + m_s * ns[:, _H:]
               + m_w * we[:, :_H] + m_e * we[:, _H:])

        pin = jnp.concatenate([agg, x0], axis=1)
        y = jax.nn.relu(mm(pin, qw1) + qb1)
        y = jax.nn.relu(mm(y, qw2) + qb2)
        y = jax.nn.relu(mm(y, qw3) + qb3)
        xp = mm(y, qw4) + qb4

        g = mm(jnp.concatenate([xp, h], axis=1), lw) + lb
        c = (c * jax.nn.sigmoid(g[:, 2 * _H:3 * _H] + 1.0)
             + jax.nn.sigmoid(g[:, :_H]) * jnp.tanh(g[:, _H:2 * _H]))
        h = jnp.tanh(c) * jax.nn.sigmoid(g[:, 3 * _H:])
        x = h

        hv = mm(h, wh) + bh
        pol_ref[s] = hv[:, 0:1]
        vrows = jnp.concatenate(
            [jnp.sum(hv[b * _N2:(b + 1) * _N2, 1:2], keepdims=True)
             for b in range(_BC)], axis=0)
        vs_ref[s] = vrows


def _loss_kernel(pol_ref, vs_ref, win_ref, act_ref,
                 loss_ref, pl_ref, vl_ref, acc_ref):
    f32 = jnp.float32
    a = act_ref[...]
    win = win_ref[...]
    col = lax.broadcasted_iota(jnp.int32, (_B, 384), 1)
    pls, vls, accs = [], [], []
    for s in range(_STEPS):
        lg = pol_ref[s]
        m361 = jnp.max(lg, axis=1, keepdims=True)
        mfull = jnp.maximum(m361, 0.0)
        se = (jnp.sum(jnp.exp(lg - mfull), axis=1, keepdims=True)
              + jnp.exp(-mfull))
        lse = mfull + jnp.log(se)
        sel = jnp.sum(jnp.where((col == a) & (col < _N2), lg, 0.0),
                      axis=1, keepdims=True)
        pls.append(jnp.mean(lse - sel, keepdims=True))
        fidx = jnp.min(jnp.where(lg == m361, col, 10 ** 6),
                       axis=1, keepdims=True)
        amax = jnp.where(m361 >= 0.0, fidx, _N2)
        accs.append(jnp.mean((amax == a).astype(f32), keepdims=True))
        val = jnp.tanh(vs_ref[s])
        vls.append(jnp.mean(jnp.square(win - val), keepdims=True))
    plv = jnp.concatenate(pls, axis=1)
    vlv = jnp.concatenate(vls, axis=1)
    accv = jnp.concatenate(accs, axis=1)
    pl_ref[...] = plv
    vl_ref[...] = vlv
    acc_ref[...] = accv
    loss_ref[...] = (jnp.mean(plv, keepdims=True)
                     + 0.01 * jnp.mean(vlv, keepdims=True))


def kernel(stones, player, winners, actions, edge_index, params):
    f32 = jnp.float32
    p = params

    pre_w, pre_b = p["pre"]["W"], p["pre"]["b"]
    ms = p["emb_stone"] @ pre_w[0][0:16]
    mp = p["emb_player"] @ pre_w[0][48:52]
    row_idx = jnp.arange(_N2, dtype=jnp.int32) // _S
    col_idx = jnp.arange(_N2, dtype=jnp.int32) % _S
    rc = (p["emb_row"][row_idx] @ pre_w[0][16:32]
          + p["emb_col"][col_idx] @ pre_w[0][32:48]
          + pre_b[0][None])
    rc_t = jnp.tile(rc, (_BC, 1))

    msg_w, msg_b = p["msg"]["W"], p["msg"]["b"]
    w1ab = jnp.concatenate([msg_w[0][:_H], msg_w[0][_H:2 * _H]], axis=1)

    def blockdiag(w):
        z = jnp.zeros((_H, _H), f32)
        return jnp.concatenate([jnp.concatenate([w, z], axis=1),
                                jnp.concatenate([z, w], axis=1)], axis=0)

    w2p, w3p, w4p = blockdiag(msg_w[1]), blockdiag(msg_w[2]), blockdiag(msg_w[3])
    mb1 = msg_b[0][None]
    mb2p = jnp.tile(msg_b[1][None], (1, 2))
    mb3p = jnp.tile(msg_b[2][None], (1, 2))
    mb4p = jnp.tile(msg_b[3][None], (1, 2))

    post_w, post_b = p["post"]["W"], p["post"]["b"]
    lw = p["lstm_W"]
    lb = p["lstm_b"][None]
    wh = jnp.concatenate([p["W_policy"], p["W_value"]], axis=1)
    bh = jnp.concatenate([p["b_policy"], p["b_value"]])[None]

    stones_r = stones.reshape(_N, 1)
    player_r = player.reshape(_N, 1)

    def cspec(shape):
        return pl.BlockSpec(shape, lambda i: (0,) * len(shape))

    grid = _B // _BC
    pol, vs = pl.pallas_call(
        _main_kernel,
        grid=(grid,),
        in_specs=[
            pl.BlockSpec((_R, 1), lambda i: (i, 0)),
            pl.BlockSpec((_R, 1), lambda i: (i, 0)),
            cspec((_R, _H)), cspec((3, _H)), cspec((2, _H)),
            cspec((_H, _H)), cspec((_H, _H)), cspec((_H, _H)),
            cspec((1, _H)), cspec((1, _H)), cspec((1, _H)),
            cspec((_H, 2 * _H)), cspec((1, _H)),
            cspec((2 * _H, 2 * _H)), cspec((2 * _H, 2 * _H)),
            cspec((2 * _H, 2 * _H)),
            cspec((1, 2 * _H)), cspec((1, 2 * _H)), cspec((1, 2 * _H)),
            cspec((2 * _H, _H)), cspec((_H, _H)), cspec((_H, _H)),
            cspec((_H, _H)),
            cspec((1, _H)), cspec((1, _H)), cspec((1, _H)), cspec((1, _H)),
            cspec((2 * _H, 4 * _H)), cspec((1, 4 * _H)),
            cspec((_H, 2)), cspec((1, 2)),
        ],
        out_specs=[
            pl.BlockSpec((_STEPS, _R, 1), lambda i: (0, i, 0)),
            pl.BlockSpec((_STEPS, _BC, 1), lambda i: (0, i, 0)),
        ],
        out_shape=[
            jax.ShapeDtypeStruct((_STEPS, _N, 1), f32),
            jax.ShapeDtypeStruct((_STEPS, _B, 1), f32),
        ],
        compiler_params=pltpu.CompilerParams(
            dimension_semantics=("parallel",)),
    )(stones_r, player_r, rc_t, ms, mp,
      pre_w[1], pre_w[2], pre_w[3],
      pre_b[1][None], pre_b[2][None], pre_b[3][None],
      w1ab, mb1, w2p, w3p, w4p, mb2p, mb3p, mb4p,
      post_w[0], post_w[1], post_w[2], post_w[3],
      post_b[0][None], post_b[1][None], post_b[2][None], post_b[3][None],
      lw, lb, wh, bh)

    pol3 = pol.reshape(_STEPS, _B, _N2)
    polp = jnp.pad(pol3, ((0, 0), (0, 0), (0, 384 - _N2)),
                   constant_values=_NEG)
    win2 = winners.reshape(_B, 1).astype(f32)
    act2 = actions.reshape(_B, 1).astype(jnp.int32)

    loss, plv, vlv, accv = pl.pallas_call(
        _loss_kernel,
        out_shape=[
            jax.ShapeDtypeStruct((1, 1), f32),
            jax.ShapeDtypeStruct((1, _STEPS), f32),
            jax.ShapeDtypeStruct((1, _STEPS), f32),
            jax.ShapeDtypeStruct((1, _STEPS), f32),
        ],
    )(polp, vs, win2, act2)

    return (loss[0, 0], plv[0], vlv[0], accv[0])


# speedup vs baseline: 14.7410x; 1.0662x over previous
import jax
import jax.numpy as jnp
from jax import lax
from jax.experimental import pallas as pl
from jax.experimental.pallas import tpu as pltpu

_B = 64
_S = 19
_N2 = _S * _S
_N = _B * _N2
_H = 64
_STEPS = 8
_BC = 8
_R = _BC * _N2
_NEG = -1e30


def _main_kernel(stones_ref, player_ref, rc_ref, ms_ref, mp_ref,
                 pw2_ref, pw3_ref, pw4_ref, pb2_ref, pb3_ref, pb4_ref,
                 w1ab_ref, mb1_ref, w2p_ref, w3p_ref,
                 mb2p_ref, mb3p_ref, b4p_ref,
                 qw1_ref, qw2_ref, qw3_ref, qw4_ref,
                 qb1_ref, qb2_ref, qb3_ref, qb4_ref,
                 lw_ref, lb_ref, wh_ref, bh_ref,
                 pol_ref, vs_ref):
    f32 = jnp.float32

    def mm(a, b):
        return jnp.dot(a, b, preferred_element_type=f32)

    s_col = stones_ref[...]
    p_col = player_ref[...]
    base = rc_ref[...]
    for k in range(3):
        base = base + (s_col == k).astype(f32) * ms_ref[k:k + 1, :]
    for k in range(2):
        base = base + (p_col == k).astype(f32) * mp_ref[k:k + 1, :]
    y = jax.nn.relu(base)
    y = jax.nn.relu(mm(y, pw2_ref[...]) + pb2_ref[...])
    y = jax.nn.relu(mm(y, pw3_ref[...]) + pb3_ref[...])
    x0 = mm(y, pw4_ref[...]) + pb4_ref[...]

    pos = lax.broadcasted_iota(jnp.int32, (_R, 1), 0) % _N2
    cp = pos % _S
    m_n = (pos >= _S).astype(f32)
    m_s = (pos < _N2 - _S).astype(f32)
    m_w = (cp >= 1).astype(f32)
    m_e = (cp <= _S - 2).astype(f32)

    z19 = jnp.zeros((_S, _H), f32)
    z1 = jnp.zeros((1, _H), f32)

    w1ab = w1ab_ref[...]
    mb1 = mb1_ref[...]
    w2p, w3p = w2p_ref[...], w3p_ref[...]
    mb2p, mb3p, b4p = mb2p_ref[...], mb3p_ref[...], b4p_ref[...]
    deg = m_n + m_s + m_w + m_e
    qw1, qw2, qw3, qw4 = qw1_ref[...], qw2_ref[...], qw3_ref[...], qw4_ref[...]
    qb1, qb2, qb3, qb4 = qb1_ref[...], qb2_ref[...], qb3_ref[...], qb4_ref[...]
    lw, lb = lw_ref[...], lb_ref[...]
    wh, bh = wh_ref[...], bh_ref[...]

    x = x0
    h = jnp.zeros((_R, _H), f32)
    c = jnp.zeros((_R, _H), f32)

    for s in range(_STEPS):
        ab = mm(x, w1ab)
        a_part = ab[:, :_H]
        b_part = ab[:, _H:] + mb1
        a_n = jnp.concatenate([z19, a_part[:-_S]], axis=0)
        a_s = jnp.concatenate([a_part[_S:], z19], axis=0)
        a_w = jnp.concatenate([z1, a_part[:-1]], axis=0)
        a_e = jnp.concatenate([a_part[1:], z1], axis=0)
        h1n = jax.nn.relu(a_n + b_part)
        h1s = jax.nn.relu(a_s + b_part)
        h1w = jax.nn.relu(a_w + b_part)
        h1e = jax.nn.relu(a_e + b_part)
        ns = jnp.concatenate([h1n, h1s], axis=1)
        ns = jax.nn.relu(mm(ns, w2p) + mb2p)
        ns = jax.nn.relu(mm(ns, w3p) + mb3p)
        we = jnp.concatenate([h1w, h1e], axis=1)
        we = jax.nn.relu(mm(we, w2p) + mb2p)
        we = jax.nn.relu(mm(we, w3p) + mb3p)
        sm = (m_n * ns[:, :_H] + m_s * ns[:, _H:]
              + m_w * we[:, :_H] + m_e * we[:, _H:])

        pin = jnp.concatenate([sm, x0], axis=1)
        y = jax.nn.relu(mm(pin, qw1) + deg * b4p + qb1)
        y = jax.nn.relu(mm(y, qw2) + qb2)
        y = jax.nn.relu(mm(y, qw3) + qb3)
        xp = mm(y, qw4) + qb4

        g = mm(jnp.concatenate([xp, h], axis=1), lw) + lb
        c = (c * jax.nn.sigmoid(g[:, 2 * _H:3 * _H] + 1.0)
             + jax.nn.sigmoid(g[:, :_H]) * jnp.tanh(g[:, _H:2 * _H]))
        h = jnp.tanh(c) * jax.nn.sigmoid(g[:, 3 * _H:])
        x = h

        hv = mm(h, wh) + bh
        pol_ref[s] = hv[:, 0:1]
        vrows = jnp.concatenate(
            [jnp.sum(hv[b * _N2:(b + 1) * _N2, 1:2], keepdims=True)
             for b in range(_BC)], axis=0)
        vs_ref[s] = vrows


def _loss_kernel(pol_ref, vs_ref, win_ref, act_ref,
                 loss_ref, pl_ref, vl_ref, acc_ref):
    f32 = jnp.float32
    a = act_ref[...]
    win = win_ref[...]
    col = lax.broadcasted_iota(jnp.int32, (_B, 384), 1)
    pls, vls, accs = [], [], []
    for s in range(_STEPS):
        lg = pol_ref[s]
        m361 = jnp.max(lg, axis=1, keepdims=True)
        mfull = jnp.maximum(m361, 0.0)
        se = (jnp.sum(jnp.exp(lg - mfull), axis=1, keepdims=True)
              + jnp.exp(-mfull))
        lse = mfull + jnp.log(se)
        sel = jnp.sum(jnp.where((col == a) & (col < _N2), lg, 0.0),
                      axis=1, keepdims=True)
        pls.append(jnp.mean(lse - sel, keepdims=True))
        fidx = jnp.min(jnp.where(lg == m361, col, 10 ** 6),
                       axis=1, keepdims=True)
        amax = jnp.where(m361 >= 0.0, fidx, _N2)
        accs.append(jnp.mean((amax == a).astype(f32), keepdims=True))
        val = jnp.tanh(vs_ref[s])
        vls.append(jnp.mean(jnp.square(win - val), keepdims=True))
    plv = jnp.concatenate(pls, axis=1)
    vlv = jnp.concatenate(vls, axis=1)
    accv = jnp.concatenate(accs, axis=1)
    pl_ref[...] = plv
    vl_ref[...] = vlv
    acc_ref[...] = accv
    loss_ref[...] = (jnp.mean(plv, keepdims=True)
                     + 0.01 * jnp.mean(vlv, keepdims=True))


def kernel(stones, player, winners, actions, edge_index, params):
    f32 = jnp.float32
    p = params

    pre_w, pre_b = p["pre"]["W"], p["pre"]["b"]
    ms = p["emb_stone"] @ pre_w[0][0:16]
    mp = p["emb_player"] @ pre_w[0][48:52]
    row_idx = jnp.arange(_N2, dtype=jnp.int32) // _S
    col_idx = jnp.arange(_N2, dtype=jnp.int32) % _S
    rc = (p["emb_row"][row_idx] @ pre_w[0][16:32]
          + p["emb_col"][col_idx] @ pre_w[0][32:48]
          + pre_b[0][None])
    rc_t = jnp.tile(rc, (_BC, 1))

    msg_w, msg_b = p["msg"]["W"], p["msg"]["b"]
    w1ab = jnp.concatenate([msg_w[0][:_H], msg_w[0][_H:2 * _H]], axis=1)

    def blockdiag(w):
        z = jnp.zeros((_H, _H), f32)
        return jnp.concatenate([jnp.concatenate([w, z], axis=1),
                                jnp.concatenate([z, w], axis=1)], axis=0)

    w2p, w3p = blockdiag(msg_w[1]), blockdiag(msg_w[2])
    mb1 = msg_b[0][None]
    mb2p = jnp.tile(msg_b[1][None], (1, 2))
    mb3p = jnp.tile(msg_b[2][None], (1, 2))

    post_w, post_b = p["post"]["W"], p["post"]["b"]
    qw1f = jnp.concatenate([msg_w[3] @ post_w[0][:_H], post_w[0][_H:]], axis=0)
    b4p = (msg_b[3] @ post_w[0][:_H])[None]
    lw = p["lstm_W"]
    lb = p["lstm_b"][None]
    wh = jnp.concatenate([p["W_policy"], p["W_value"]], axis=1)
    bh = jnp.concatenate([p["b_policy"], p["b_value"]])[None]

    stones_r = stones.reshape(_N, 1)
    player_r = player.reshape(_N, 1)

    def cspec(shape):
        return pl.BlockSpec(shape, lambda i: (0,) * len(shape))

    grid = _B // _BC
    pol, vs = pl.pallas_call(
        _main_kernel,
        grid=(grid,),
        in_specs=[
            pl.BlockSpec((_R, 1), lambda i: (i, 0)),
            pl.BlockSpec((_R, 1), lambda i: (i, 0)),
            cspec((_R, _H)), cspec((3, _H)), cspec((2, _H)),
            cspec((_H, _H)), cspec((_H, _H)), cspec((_H, _H)),
            cspec((1, _H)), cspec((1, _H)), cspec((1, _H)),
            cspec((_H, 2 * _H)), cspec((1, _H)),
            cspec((2 * _H, 2 * _H)), cspec((2 * _H, 2 * _H)),
            cspec((1, 2 * _H)), cspec((1, 2 * _H)), cspec((1, _H)),
            cspec((2 * _H, _H)), cspec((_H, _H)), cspec((_H, _H)),
            cspec((_H, _H)),
            cspec((1, _H)), cspec((1, _H)), cspec((1, _H)), cspec((1, _H)),
            cspec((2 * _H, 4 * _H)), cspec((1, 4 * _H)),
            cspec((_H, 2)), cspec((1, 2)),
        ],
        out_specs=[
            pl.BlockSpec((_STEPS, _R, 1), lambda i: (0, i, 0)),
            pl.BlockSpec((_STEPS, _BC, 1), lambda i: (0, i, 0)),
        ],
        out_shape=[
            jax.ShapeDtypeStruct((_STEPS, _N, 1), f32),
            jax.ShapeDtypeStruct((_STEPS, _B, 1), f32),
        ],
        compiler_params=pltpu.CompilerParams(
            dimension_semantics=("parallel",)),
    )(stones_r, player_r, rc_t, ms, mp,
      pre_w[1], pre_w[2], pre_w[3],
      pre_b[1][None], pre_b[2][None], pre_b[3][None],
      w1ab, mb1, w2p, w3p, mb2p, mb3p, b4p,
      qw1f, post_w[1], post_w[2], post_w[3],
      post_b[0][None], post_b[1][None], post_b[2][None], post_b[3][None],
      lw, lb, wh, bh)

    pol3 = pol.reshape(_STEPS, _B, _N2)
    polp = jnp.pad(pol3, ((0, 0), (0, 0), (0, 384 - _N2)),
                   constant_values=_NEG)
    win2 = winners.reshape(_B, 1).astype(f32)
    act2 = actions.reshape(_B, 1).astype(jnp.int32)

    loss, plv, vlv, accv = pl.pallas_call(
        _loss_kernel,
        out_shape=[
            jax.ShapeDtypeStruct((1, 1), f32),
            jax.ShapeDtypeStruct((1, _STEPS), f32),
            jax.ShapeDtypeStruct((1, _STEPS), f32),
            jax.ShapeDtypeStruct((1, _STEPS), f32),
        ],
    )(polp, vs, win2, act2)

    return (loss[0, 0], plv[0], vlv[0], accv[0])


# speedup vs baseline: 14.9413x; 1.0136x over previous
import jax
import jax.numpy as jnp
from jax import lax
from jax.experimental import pallas as pl
from jax.experimental.pallas import tpu as pltpu

_B = 64
_S = 19
_N2 = _S * _S
_N = _B * _N2
_H = 64
_STEPS = 8
_BC = 8
_R = _BC * _N2
_NEG = -1e30


def _main_kernel(stones_ref, player_ref, rc_ref, ms_ref, mp_ref,
                 pw2_ref, pw3_ref, pw4_ref, pb2_ref, pb3_ref, pb4_ref,
                 w1ab_ref, mb1_ref, w2p_ref, w3p_ref,
                 mb2p_ref, mb3p_ref, b4p_ref,
                 qw1_ref, qw2_ref, qw3_ref, qw4_ref,
                 qb1_ref, qb2_ref, qb3_ref, qb4_ref,
                 lw_ref, lb_ref, wh_ref, bh_ref,
                 pol_ref, vs_ref):
    f32 = jnp.float32

    def mm(a, b):
        return jnp.dot(a, b, preferred_element_type=f32)

    s_col = stones_ref[...]
    p_col = player_ref[...]
    base = rc_ref[...]
    for k in range(3):
        base = base + (s_col == k).astype(f32) * ms_ref[k:k + 1, :]
    for k in range(2):
        base = base + (p_col == k).astype(f32) * mp_ref[k:k + 1, :]
    y = jax.nn.relu(base)
    y = jax.nn.relu(mm(y, pw2_ref[...]) + pb2_ref[...])
    y = jax.nn.relu(mm(y, pw3_ref[...]) + pb3_ref[...])
    x0 = mm(y, pw4_ref[...]) + pb4_ref[...]

    pos = lax.broadcasted_iota(jnp.int32, (_R, 1), 0) % _N2
    cp = pos % _S
    m_n = (pos >= _S).astype(f32)
    m_s = (pos < _N2 - _S).astype(f32)
    m_w = (cp >= 1).astype(f32)
    m_e = (cp <= _S - 2).astype(f32)

    z19 = jnp.zeros((_S, _H), f32)
    z1 = jnp.zeros((1, _H), f32)

    w1ab = w1ab_ref[...]
    mb1 = mb1_ref[...]
    w2p, w3p = w2p_ref[...], w3p_ref[...]
    mb2p, mb3p, b4p = mb2p_ref[...], mb3p_ref[...], b4p_ref[...]
    deg = m_n + m_s + m_w + m_e
    qw1, qw2, qw3, qw4 = qw1_ref[...], qw2_ref[...], qw3_ref[...], qw4_ref[...]
    qb1, qb2, qb3, qb4 = qb1_ref[...], qb2_ref[...], qb3_ref[...], qb4_ref[...]
    lw, lb = lw_ref[...], lb_ref[...]
    wh, bh = wh_ref[...], bh_ref[...]

    x = x0
    h = jnp.zeros((_R, _H), f32)
    c = jnp.zeros((_R, _H), f32)

    for s in range(_STEPS):
        ab = mm(x, w1ab)
        a_part = ab[:, :_H]
        b_part = ab[:, _H:] + mb1
        a_n = jnp.concatenate([z19, a_part[:-_S]], axis=0)
        a_s = jnp.concatenate([a_part[_S:], z19], axis=0)
        a_w = jnp.concatenate([z1, a_part[:-1]], axis=0)
        a_e = jnp.concatenate([a_part[1:], z1], axis=0)
        h1n = jax.nn.relu(a_n + b_part)
        h1s = jax.nn.relu(a_s + b_part)
        h1w = jax.nn.relu(a_w + b_part)
        h1e = jax.nn.relu(a_e + b_part)
        q = jnp.concatenate([h1n, h1s, h1w, h1e], axis=1)
        q = jax.nn.relu(mm(q, w2p) + mb2p)
        q = jax.nn.relu(mm(q, w3p) + mb3p)
        sm = (m_n * q[:, :_H] + m_s * q[:, _H:2 * _H]
              + m_w * q[:, 2 * _H:3 * _H] + m_e * q[:, 3 * _H:])

        pin = jnp.concatenate([sm, x0], axis=1)
        y = jax.nn.relu(mm(pin, qw1) + deg * b4p + qb1)
        y = jax.nn.relu(mm(y, qw2) + qb2)
        y = jax.nn.relu(mm(y, qw3) + qb3)
        xp = mm(y, qw4) + qb4

        g = mm(jnp.concatenate([xp, h], axis=1), lw) + lb
        c = (c * jax.nn.sigmoid(g[:, 2 * _H:3 * _H] + 1.0)
             + jax.nn.sigmoid(g[:, :_H]) * jnp.tanh(g[:, _H:2 * _H]))
        h = jnp.tanh(c) * jax.nn.sigmoid(g[:, 3 * _H:])
        x = h

        hv = mm(h, wh) + bh
        pol_ref[s] = hv[:, 0:1]
        vrows = jnp.concatenate(
            [jnp.sum(hv[b * _N2:(b + 1) * _N2, 1:2], keepdims=True)
             for b in range(_BC)], axis=0)
        vs_ref[s] = vrows


def _loss_kernel(pol_ref, vs_ref, win_ref, act_ref,
                 loss_ref, pl_ref, vl_ref, acc_ref):
    f32 = jnp.float32
    a = act_ref[...]
    win = win_ref[...]
    col = lax.broadcasted_iota(jnp.int32, (_B, 384), 1)
    pls, vls, accs = [], [], []
    for s in range(_STEPS):
        lg = pol_ref[s]
        m361 = jnp.max(lg, axis=1, keepdims=True)
        mfull = jnp.maximum(m361, 0.0)
        se = (jnp.sum(jnp.exp(lg - mfull), axis=1, keepdims=True)
              + jnp.exp(-mfull))
        lse = mfull + jnp.log(se)
        sel = jnp.sum(jnp.where((col == a) & (col < _N2), lg, 0.0),
                      axis=1, keepdims=True)
        pls.append(jnp.mean(lse - sel, keepdims=True))
        fidx = jnp.min(jnp.where(lg == m361, col, 10 ** 6),
                       axis=1, keepdims=True)
        amax = jnp.where(m361 >= 0.0, fidx, _N2)
        accs.append(jnp.mean((amax == a).astype(f32), keepdims=True))
        val = jnp.tanh(vs_ref[s])
        vls.append(jnp.mean(jnp.square(win - val), keepdims=True))
    plv = jnp.concatenate(pls, axis=1)
    vlv = jnp.concatenate(vls, axis=1)
    accv = jnp.concatenate(accs, axis=1)
    pl_ref[...] = plv
    vl_ref[...] = vlv
    acc_ref[...] = accv
    loss_ref[...] = (jnp.mean(plv, keepdims=True)
                     + 0.01 * jnp.mean(vlv, keepdims=True))


def kernel(stones, player, winners, actions, edge_index, params):
    f32 = jnp.float32
    p = params

    pre_w, pre_b = p["pre"]["W"], p["pre"]["b"]
    ms = p["emb_stone"] @ pre_w[0][0:16]
    mp = p["emb_player"] @ pre_w[0][48:52]
    row_idx = jnp.arange(_N2, dtype=jnp.int32) // _S
    col_idx = jnp.arange(_N2, dtype=jnp.int32) % _S
    rc = (p["emb_row"][row_idx] @ pre_w[0][16:32]
          + p["emb_col"][col_idx] @ pre_w[0][32:48]
          + pre_b[0][None])
    rc_t = jnp.tile(rc, (_BC, 1))

    msg_w, msg_b = p["msg"]["W"], p["msg"]["b"]
    w1ab = jnp.concatenate([msg_w[0][:_H], msg_w[0][_H:2 * _H]], axis=1)

    def blockdiag4(w):
        out = jnp.zeros((4 * _H, 4 * _H), f32)
        for k in range(4):
            out = out.at[k * _H:(k + 1) * _H, k * _H:(k + 1) * _H].set(w)
        return out

    w2p, w3p = blockdiag4(msg_w[1]), blockdiag4(msg_w[2])
    mb1 = msg_b[0][None]
    mb2p = jnp.tile(msg_b[1][None], (1, 4))
    mb3p = jnp.tile(msg_b[2][None], (1, 4))

    post_w, post_b = p["post"]["W"], p["post"]["b"]
    qw1f = jnp.concatenate([msg_w[3] @ post_w[0][:_H], post_w[0][_H:]], axis=0)
    b4p = (msg_b[3] @ post_w[0][:_H])[None]
    lw = p["lstm_W"]
    lb = p["lstm_b"][None]
    wh = jnp.concatenate([p["W_policy"], p["W_value"]], axis=1)
    bh = jnp.concatenate([p["b_policy"], p["b_value"]])[None]

    stones_r = stones.reshape(_N, 1)
    player_r = player.reshape(_N, 1)

    def cspec(shape):
        return pl.BlockSpec(shape, lambda i: (0,) * len(shape))

    grid = _B // _BC
    pol, vs = pl.pallas_call(
        _main_kernel,
        grid=(grid,),
        in_specs=[
            pl.BlockSpec((_R, 1), lambda i: (i, 0)),
            pl.BlockSpec((_R, 1), lambda i: (i, 0)),
            cspec((_R, _H)), cspec((3, _H)), cspec((2, _H)),
            cspec((_H, _H)), cspec((_H, _H)), cspec((_H, _H)),
            cspec((1, _H)), cspec((1, _H)), cspec((1, _H)),
            cspec((_H, 2 * _H)), cspec((1, _H)),
            cspec((4 * _H, 4 * _H)), cspec((4 * _H, 4 * _H)),
            cspec((1, 4 * _H)), cspec((1, 4 * _H)), cspec((1, _H)),
            cspec((2 * _H, _H)), cspec((_H, _H)), cspec((_H, _H)),
            cspec((_H, _H)),
            cspec((1, _H)), cspec((1, _H)), cspec((1, _H)), cspec((1, _H)),
            cspec((2 * _H, 4 * _H)), cspec((1, 4 * _H)),
            cspec((_H, 2)), cspec((1, 2)),
        ],
        out_specs=[
            pl.BlockSpec((_STEPS, _R, 1), lambda i: (0, i, 0)),
            pl.BlockSpec((_STEPS, _BC, 1), lambda i: (0, i, 0)),
        ],
        out_shape=[
            jax.ShapeDtypeStruct((_STEPS, _N, 1), f32),
            jax.ShapeDtypeStruct((_STEPS, _B, 1), f32),
        ],
        compiler_params=pltpu.CompilerParams(
            dimension_semantics=("parallel",)),
    )(stones_r, player_r, rc_t, ms, mp,
      pre_w[1], pre_w[2], pre_w[3],
      pre_b[1][None], pre_b[2][None], pre_b[3][None],
      w1ab, mb1, w2p, w3p, mb2p, mb3p, b4p,
      qw1f, post_w[1], post_w[2], post_w[3],
      post_b[0][None], post_b[1][None], post_b[2][None], post_b[3][None],
      lw, lb, wh, bh)

    pol3 = pol.reshape(_STEPS, _B, _N2)
    polp = jnp.pad(pol3, ((0, 0), (0, 0), (0, 384 - _N2)),
                   constant_values=_NEG)
    win2 = winners.reshape(_B, 1).astype(f32)
    act2 = actions.reshape(_B, 1).astype(jnp.int32)

    loss, plv, vlv, accv = pl.pallas_call(
        _loss_kernel,
        out_shape=[
            jax.ShapeDtypeStruct((1, 1), f32),
            jax.ShapeDtypeStruct((1, _STEPS), f32),
            jax.ShapeDtypeStruct((1, _STEPS), f32),
            jax.ShapeDtypeStruct((1, _STEPS), f32),
        ],
    )(polp, vs, win2, act2)

    return (loss[0, 0], plv[0], vlv[0], accv[0])
